# baseline (speedup 1.0000x reference)
import jax
import jax.numpy as jnp
from jax.experimental import pallas as pl
from jax.experimental.pallas import tpu as pltpu

_EPS = 1e-05
_TB = 512


def _cgn_kernel(x_ref, w_ref, o_ref, carry_ref):
    t_idx = pl.program_id(1)

    @pl.when(t_idx == 0)
    def _():
        carry_ref[0] = 0.0
        carry_ref[1] = 0.0

    x = x_ref[0]
    tb, c = x.shape

    row = jax.lax.broadcasted_iota(jnp.int32, (tb, tb), 0)
    col = jax.lax.broadcasted_iota(jnp.int32, (tb, tb), 1)
    ltri = (col <= row).astype(jnp.float32)

    s = jnp.sum(x, axis=1, keepdims=True)
    cum_s = jax.lax.dot_general(
        ltri, s, (((1,), (0,)), ((), ())),
        precision=jax.lax.Precision.DEFAULT,
    ) + carry_ref[0]

    tpos = jax.lax.broadcasted_iota(jnp.int32, (tb, 1), 0)
    count = (t_idx * tb + 1 + tpos).astype(jnp.float32) * jnp.float32(c)
    mean = cum_s / count

    xm = x - mean
    d = jnp.sum(xm * xm, axis=1, keepdims=True)
    cum_d = jax.lax.dot_general(
        ltri, d, (((1,), (0,)), ((), ())),
        precision=jax.lax.Precision.DEFAULT,
    ) + carry_ref[1]

    inv = jax.lax.rsqrt(cum_d / count + _EPS)
    o_ref[0] = xm * inv * w_ref[...]

    carry_ref[0] = carry_ref[0] + jnp.sum(s)
    carry_ref[1] = carry_ref[1] + jnp.sum(d)


def kernel(x, weight):
    input_dtype = x.dtype
    xf = x.astype(jnp.float32)
    B, T, C = xf.shape
    w2 = weight.astype(jnp.float32).reshape(1, C)

    out = pl.pallas_call(
        _cgn_kernel,
        out_shape=jax.ShapeDtypeStruct((B, T, C), jnp.float32),
        grid=(B, T // _TB),
        in_specs=[
            pl.BlockSpec((1, _TB, C), lambda b, t: (b, t, 0)),
            pl.BlockSpec((1, C), lambda b, t: (0, 0)),
        ],
        out_specs=pl.BlockSpec((1, _TB, C), lambda b, t: (b, t, 0)),
        scratch_shapes=[pltpu.SMEM((2,), jnp.float32)],
        compiler_params=pltpu.CompilerParams(
            dimension_semantics=("parallel", "arbitrary"),
        ),
        name="cumulative_group_norm",
    )(xf, w2)
    return out.astype(input_dtype)


# speedup vs baseline: 1.6472x; 1.6472x over previous
import jax
import jax.numpy as jnp
from jax.experimental import pallas as pl
from jax.experimental.pallas import tpu as pltpu

_EPS = 1e-05
_TB = 512
_GB = 4


def _cgn_kernel(x_ref, w_ref, o_ref, carry_ref, ltri_ref):
    t_idx = pl.program_id(1)

    @pl.when(t_idx == 0)
    def _():
        for g in range(_GB):
            carry_ref[g, 0] = 0.0
            carry_ref[g, 1] = 0.0
        row = jax.lax.broadcasted_iota(jnp.int32, (_TB, _TB), 0)
        col = jax.lax.broadcasted_iota(jnp.int32, (_TB, _TB), 1)
        ltri_ref[...] = (col <= row).astype(jnp.float32).astype(jnp.bfloat16)

    ltri = ltri_ref[...]
    w = w_ref[...]
    tpos = jax.lax.broadcasted_iota(jnp.int32, (_TB, 1), 0)

    for g in range(_GB):
        x = x_ref[g]
        tb, c = x.shape

        s = jnp.sum(x, axis=1, keepdims=True)
        cum_s = jax.lax.dot_general(
            ltri, s.astype(jnp.bfloat16), (((1,), (0,)), ((), ())),
            preferred_element_type=jnp.float32,
        ) + carry_ref[g, 0]

        count = (t_idx * tb + 1 + tpos).astype(jnp.float32) * jnp.float32(c)
        mean = cum_s / count

        xm = x - mean
        d = jnp.sum(xm * xm, axis=1, keepdims=True)
        cum_d = jax.lax.dot_general(
            ltri, d.astype(jnp.bfloat16), (((1,), (0,)), ((), ())),
            preferred_element_type=jnp.float32,
        ) + carry_ref[g, 1]

        inv = jax.lax.rsqrt(cum_d / count + _EPS)
        o_ref[g] = xm * inv * w

        carry_ref[g, 0] = carry_ref[g, 0] + jnp.sum(s)
        carry_ref[g, 1] = carry_ref[g, 1] + jnp.sum(d)


def kernel(x, weight):
    input_dtype = x.dtype
    xf = x.astype(jnp.float32)
    B, T, C = xf.shape
    w2 = weight.astype(jnp.float32).reshape(1, C)

    out = pl.pallas_call(
        _cgn_kernel,
        out_shape=jax.ShapeDtypeStruct((B, T, C), jnp.float32),
        grid=(B // _GB, T // _TB),
        in_specs=[
            pl.BlockSpec((_GB, _TB, C), lambda b, t: (b, t, 0)),
            pl.BlockSpec((1, C), lambda b, t: (0, 0)),
        ],
        out_specs=pl.BlockSpec((_GB, _TB, C), lambda b, t: (b, t, 0)),
        scratch_shapes=[
            pltpu.SMEM((_GB, 2), jnp.float32),
            pltpu.VMEM((_TB, _TB), jnp.bfloat16),
        ],
        compiler_params=pltpu.CompilerParams(
            dimension_semantics=("parallel", "arbitrary"),
            vmem_limit_bytes=56 * 1024 * 1024,
        ),
        name="cumulative_group_norm",
    )(xf, w2)
    return out.astype(input_dtype)


# speedup vs baseline: 1.6731x; 1.0157x over previous
import jax
import jax.numpy as jnp
from jax.experimental import pallas as pl
from jax.experimental.pallas import tpu as pltpu

_EPS = 1e-05
_TB = 512
_GB = 8


def _cgn_kernel(x_ref, w_ref, o_ref, carry_ref, ltri_ref):
    t_idx = pl.program_id(1)

    @pl.when(t_idx == 0)
    def _():
        for g in range(_GB):
            carry_ref[g, 0] = 0.0
            carry_ref[g, 1] = 0.0
        row = jax.lax.broadcasted_iota(jnp.int32, (_TB, _TB), 0)
        col = jax.lax.broadcasted_iota(jnp.int32, (_TB, _TB), 1)
        ltri_ref[...] = (col <= row).astype(jnp.float32).astype(jnp.bfloat16)

    ltri = ltri_ref[...]
    w = w_ref[...]
    tpos = jax.lax.broadcasted_iota(jnp.int32, (_TB, 1), 0)

    for g in range(_GB):
        x = x_ref[g]
        tb, c = x.shape

        s = jnp.sum(x, axis=1, keepdims=True)
        cum_s = jax.lax.dot_general(
            ltri, s.astype(jnp.bfloat16), (((1,), (0,)), ((), ())),
            preferred_element_type=jnp.float32,
        ) + carry_ref[g, 0]

        count = (t_idx * tb + 1 + tpos).astype(jnp.float32) * jnp.float32(c)
        mean = cum_s / count

        xm = x - mean
        d = jnp.sum(xm * xm, axis=1, keepdims=True)
        cum_d = jax.lax.dot_general(
            ltri, d.astype(jnp.bfloat16), (((1,), (0,)), ((), ())),
            preferred_element_type=jnp.float32,
        ) + carry_ref[g, 1]

        inv = jax.lax.rsqrt(cum_d / count + _EPS)
        o_ref[g] = xm * inv * w

        carry_ref[g, 0] = carry_ref[g, 0] + jnp.sum(s)
        carry_ref[g, 1] = carry_ref[g, 1] + jnp.sum(d)


def kernel(x, weight):
    input_dtype = x.dtype
    xf = x.astype(jnp.float32)
    B, T, C = xf.shape
    w2 = weight.astype(jnp.float32).reshape(1, C)

    out = pl.pallas_call(
        _cgn_kernel,
        out_shape=jax.ShapeDtypeStruct((B, T, C), jnp.float32),
        grid=(B // _GB, T // _TB),
        in_specs=[
            pl.BlockSpec((_GB, _TB, C), lambda b, t: (b, t, 0)),
            pl.BlockSpec((1, C), lambda b, t: (0, 0)),
        ],
        out_specs=pl.BlockSpec((_GB, _TB, C), lambda b, t: (b, t, 0)),
        scratch_shapes=[
            pltpu.SMEM((_GB, 2), jnp.float32),
            pltpu.VMEM((_TB, _TB), jnp.bfloat16),
        ],
        compiler_params=pltpu.CompilerParams(
            dimension_semantics=("parallel", "arbitrary"),
            vmem_limit_bytes=56 * 1024 * 1024,
        ),
        name="cumulative_group_norm",
    )(xf, w2)
    return out.astype(input_dtype)


# speedup vs baseline: 2.0485x; 1.2244x over previous
import jax
import jax.numpy as jnp
from jax.experimental import pallas as pl
from jax.experimental.pallas import tpu as pltpu

_EPS = 1e-05
_TB = 512
_GB = 8


def _cgn_kernel(x_ref, w_ref, o_ref, carry_ref, ltri_ref):
    t_idx = pl.program_id(1)

    @pl.when(t_idx == 0)
    def _():
        carry_ref[...] = jnp.zeros_like(carry_ref)
        row = jax.lax.broadcasted_iota(jnp.int32, (_TB, _TB), 0)
        col = jax.lax.broadcasted_iota(jnp.int32, (_TB, _TB), 1)
        ltri_ref[...] = (col <= row).astype(jnp.float32).astype(jnp.bfloat16)

    ltri = ltri_ref[...]
    w = w_ref[...]
    cf = jnp.float32(x_ref.shape[2])

    tpos = jax.lax.broadcasted_iota(jnp.int32, (_TB, 1), 0)
    count = (t_idx * _TB + 1 + tpos).astype(jnp.float32) * cf
    rcp_count = 1.0 / count

    cols = []
    for g in range(_GB):
        x = x_ref[g]
        cols.append(jnp.sum(x, axis=1, keepdims=True))
    for g in range(_GB):
        x = x_ref[g]
        cols.append(jnp.sum(x * x, axis=1, keepdims=True))
    sq = jnp.concatenate(cols, axis=1)

    cum_sq = jax.lax.dot_general(
        ltri, sq.astype(jnp.bfloat16), (((1,), (0,)), ((), ())),
        preferred_element_type=jnp.float32,
    )

    s_all = sq[:, 0:_GB]
    cum_s = cum_sq[:, 0:_GB] + carry_ref[0:1, 0:_GB]
    cum_q = cum_sq[:, _GB:2 * _GB] + carry_ref[1:2, 0:_GB]

    mean = cum_s * rcp_count
    gg = mean * (2.0 * s_all - cf * mean)
    cum_g = jax.lax.dot_general(
        ltri, gg.astype(jnp.bfloat16), (((1,), (0,)), ((), ())),
        preferred_element_type=jnp.float32,
    ) + carry_ref[2:3, 0:_GB]

    inv = jax.lax.rsqrt((cum_q - cum_g) * rcp_count + _EPS)

    for g in range(_GB):
        x = x_ref[g]
        o_ref[g] = (x - mean[:, g:g + 1]) * inv[:, g:g + 1] * w

    carry_ref[0:1, 0:_GB] = carry_ref[0:1, 0:_GB] + jnp.sum(
        s_all, axis=0, keepdims=True)
    carry_ref[1:2, 0:_GB] = carry_ref[1:2, 0:_GB] + jnp.sum(
        sq[:, _GB:2 * _GB], axis=0, keepdims=True)
    carry_ref[2:3, 0:_GB] = carry_ref[2:3, 0:_GB] + jnp.sum(
        gg, axis=0, keepdims=True)


def kernel(x, weight):
    input_dtype = x.dtype
    xf = x.astype(jnp.float32)
    B, T, C = xf.shape
    w2 = weight.astype(jnp.float32).reshape(1, C)

    out = pl.pallas_call(
        _cgn_kernel,
        out_shape=jax.ShapeDtypeStruct((B, T, C), jnp.float32),
        grid=(B // _GB, T // _TB),
        in_specs=[
            pl.BlockSpec((_GB, _TB, C), lambda b, t: (b, t, 0)),
            pl.BlockSpec((1, C), lambda b, t: (0, 0)),
        ],
        out_specs=pl.BlockSpec((_GB, _TB, C), lambda b, t: (b, t, 0)),
        scratch_shapes=[
            pltpu.VMEM((8, 128), jnp.float32),
            pltpu.VMEM((_TB, _TB), jnp.bfloat16),
        ],
        compiler_params=pltpu.CompilerParams(
            dimension_semantics=("parallel", "arbitrary"),
            vmem_limit_bytes=56 * 1024 * 1024,
        ),
        name="cumulative_group_norm",
    )(xf, w2)
    return out.astype(input_dtype)


# speedup vs baseline: 2.0650x; 1.0080x over previous
import jax
import jax.numpy as jnp
from jax.experimental import pallas as pl
from jax.experimental.pallas import tpu as pltpu

_EPS = 1e-05
_TB = 512
_GB = 8


def _cgn_kernel(x_ref, w_ref, o_ref, carry_ref, ltri_ref):
    t_idx = pl.program_id(1)

    @pl.when(t_idx == 0)
    def _():
        carry_ref[...] = jnp.zeros_like(carry_ref)
        row = jax.lax.broadcasted_iota(jnp.int32, (_TB, _TB), 0)
        col = jax.lax.broadcasted_iota(jnp.int32, (_TB, _TB), 1)
        ltri_ref[...] = (col <= row).astype(jnp.float32).astype(jnp.bfloat16)

    ltri = ltri_ref[...]
    w = w_ref[...]
    cf = jnp.float32(x_ref.shape[2])

    tpos = jax.lax.broadcasted_iota(jnp.int32, (_TB, 1), 0)
    count = (t_idx * _TB + 1 + tpos).astype(jnp.float32) * cf
    rcp_count = 1.0 / count

    s_cols = []
    q_cols = []
    for g in range(_GB):
        x = x_ref[g]
        s_cols.append(jnp.sum(x, axis=1, keepdims=True))
        q_cols.append(jnp.sum(x * x, axis=1, keepdims=True))
    sq = jnp.concatenate(s_cols + q_cols, axis=1)

    cum_sq = jax.lax.dot_general(
        ltri, sq.astype(jnp.bfloat16), (((1,), (0,)), ((), ())),
        preferred_element_type=jnp.float32,
    )

    s_all = sq[:, 0:_GB]
    cum_s = cum_sq[:, 0:_GB] + carry_ref[0:1, 0:_GB]
    cum_q = cum_sq[:, _GB:2 * _GB] + carry_ref[1:2, 0:_GB]

    mean = cum_s * rcp_count
    gg = mean * (2.0 * s_all - cf * mean)
    cum_g = jax.lax.dot_general(
        ltri, gg.astype(jnp.bfloat16), (((1,), (0,)), ((), ())),
        preferred_element_type=jnp.float32,
    ) + carry_ref[2:3, 0:_GB]

    inv = jax.lax.rsqrt((cum_q - cum_g) * rcp_count + _EPS)

    for g in range(_GB):
        x = x_ref[g]
        o_ref[g] = (x - mean[:, g:g + 1]) * inv[:, g:g + 1] * w

    carry_ref[0:1, 0:_GB] = carry_ref[0:1, 0:_GB] + jnp.sum(
        s_all, axis=0, keepdims=True)
    carry_ref[1:2, 0:_GB] = carry_ref[1:2, 0:_GB] + jnp.sum(
        sq[:, _GB:2 * _GB], axis=0, keepdims=True)
    carry_ref[2:3, 0:_GB] = carry_ref[2:3, 0:_GB] + jnp.sum(
        gg, axis=0, keepdims=True)


def kernel(x, weight):
    input_dtype = x.dtype
    xf = x.astype(jnp.float32)
    B, T, C = xf.shape
    w2 = weight.astype(jnp.float32).reshape(1, C)

    out = pl.pallas_call(
        _cgn_kernel,
        out_shape=jax.ShapeDtypeStruct((B, T, C), jnp.float32),
        grid=(B // _GB, T // _TB),
        in_specs=[
            pl.BlockSpec((_GB, _TB, C), lambda b, t: (b, t, 0)),
            pl.BlockSpec((1, C), lambda b, t: (0, 0)),
        ],
        out_specs=pl.BlockSpec((_GB, _TB, C), lambda b, t: (b, t, 0)),
        scratch_shapes=[
            pltpu.VMEM((8, 128), jnp.float32),
            pltpu.VMEM((_TB, _TB), jnp.bfloat16),
        ],
        compiler_params=pltpu.CompilerParams(
            dimension_semantics=("parallel", "arbitrary"),
            vmem_limit_bytes=56 * 1024 * 1024,
        ),
        name="cumulative_group_norm",
    )(xf, w2)
    return out.astype(input_dtype)
